# baseline (speedup 1.0000x reference)
import functools

import jax
import jax.numpy as jnp
from jax import lax
from jax.experimental import pallas as pl
from jax.experimental.pallas import tpu as pltpu
from jax.experimental.pallas import tpu_sc as plsc

N = 10000
NP = 10240
E = 320000
D = 128
NC, NS = 2, 16
DUMMY = N
RPT = NP // NS

KA = 80
KC = 320
CH = 64
BR = 256
NB = NP // BR

_mesh = plsc.VectorSubcoreMesh(core_axis_name="c", subcore_axis_name="s")


@functools.partial(
    pl.kernel,
    out_type=jax.ShapeDtypeStruct((NP, D), jnp.float32),
    mesh=_mesh,
    scratch_types=[
        pltpu.VMEM((8, 64), jnp.int32),
        pltpu.VMEM((64, D), jnp.float32),
        pltpu.SemaphoreType.DMA,
    ],
)
def _xcor_kernel(permidx_hbm, x_hbm, xcor_hbm, idx_v, pbuf, psem):
    c = lax.axis_index("c")
    s = lax.axis_index("s")
    w = c * NS + s
    pltpu.sync_copy(permidx_hbm.at[w], idx_v)
    base = w * 5 * 64

    def pbody(j, carry):
        pltpu.async_copy(x_hbm.at[idx_v.at[j]], pbuf, psem).wait()
        pltpu.sync_copy(pbuf, xcor_hbm.at[pl.ds(base + j * 64, 64)])
        return carry

    lax.fori_loop(0, 5, pbody, 0)


NSUP = KC // 8


@functools.partial(
    pl.kernel,
    out_type=jax.ShapeDtypeStruct((NC * NP, D), jnp.float32),
    mesh=_mesh,
    scratch_types=[
        pltpu.VMEM((2, 8, CH), jnp.int32),
        pltpu.VMEM((2, 8, CH), jnp.int32),
        pltpu.VMEM((CH, D), jnp.float32),
        pltpu.VMEM((CH, D), jnp.float32),
        pltpu.VMEM_SHARED((NP, D), jnp.float32),
        pltpu.SemaphoreType.DMA,
        pltpu.SemaphoreType.DMA,
        pltpu.SemaphoreType.DMA,
        pltpu.SemaphoreType.DMA,
    ],
)
def _spmm_kernel(table_hbm, srcidx_hbm, dstidx_hbm, out_hbm,
                 idxs2, idxd2, buf0, buf1, acc_sh, sem0, sem1, semi0, semi1):
    c = lax.axis_index("c")
    s = lax.axis_index("s")
    w = c * NS + s
    sbase = w * KC
    dbase = s * KC
    row0 = s * RPT

    def zbody(r, carry):
        for k in range(D // 16):
            buf0[r, pl.ds(k * 16, 16)] = jnp.zeros((16,), jnp.float32)
        return carry

    lax.fori_loop(0, CH, zbody, 0)
    for j in range(RPT // CH):
        pltpu.sync_copy(buf0, acc_sh.at[pl.ds(row0 + j * CH, CH)])
    plsc.subcore_barrier()

    pltpu.sync_copy(srcidx_hbm.at[pl.ds(sbase, 8)], idxs2.at[0])
    pltpu.sync_copy(dstidx_hbm.at[pl.ds(dbase, 8)], idxd2.at[0])
    pltpu.async_copy(table_hbm.at[idxs2.at[0, 0]], buf0, sem0)

    def do_super(b, slot, nxt, last):
        if not last:
            pltpu.async_copy(srcidx_hbm.at[pl.ds(sbase + 8 * b + 8, 8)],
                             idxs2.at[nxt], semi0)
            pltpu.async_copy(dstidx_hbm.at[pl.ds(dbase + 8 * b + 8, 8)],
                             idxd2.at[nxt], semi1)
        for k in range(8):
            bufc, semc = (buf0, sem0) if k % 2 == 0 else (buf1, sem1)
            bufn, semn = (buf1, sem1) if k % 2 == 0 else (buf0, sem0)
            pltpu.make_async_copy(table_hbm.at[idxs2.at[slot, k]],
                                  bufc, semc).wait()
            if k == 6 and not last:
                pltpu.make_async_copy(srcidx_hbm.at[pl.ds(sbase + 8 * b + 8, 8)],
                                      idxs2.at[nxt], semi0).wait()
                pltpu.make_async_copy(dstidx_hbm.at[pl.ds(dbase + 8 * b + 8, 8)],
                                      idxd2.at[nxt], semi1).wait()
            if k < 7:
                pltpu.async_copy(table_hbm.at[idxs2.at[slot, k + 1]], bufn, semn)
            elif not last:
                pltpu.async_copy(table_hbm.at[idxs2.at[nxt, 0]], bufn, semn)
            pltpu.sync_copy(bufc, acc_sh.at[idxd2.at[slot, k]], add=True)

    def body(b, carry):
        slot = b % 2
        do_super(b, slot, 1 - slot, last=False)
        return carry

    lax.fori_loop(0, NSUP - 1, body, 0)
    do_super(NSUP - 1, (NSUP - 1) % 2, NSUP % 2, last=True)
    plsc.subcore_barrier()
    pltpu.sync_copy(acc_sh.at[pl.ds(row0, RPT)],
                    out_hbm.at[pl.ds(c * NP + row0, RPT)])


def _dis_from(deg_ref):
    return lax.rsqrt(deg_ref[:, 0] + 1.0)[:, None]


def _dense1_body(xp_ref, xc_ref, w1_ref, deg_ref, t1_ref):
    dis = _dis_from(deg_ref)
    t1_ref[0] = dis * jnp.dot(xp_ref[...], w1_ref[...],
                              preferred_element_type=jnp.float32)
    t1_ref[1] = dis * jnp.dot(xc_ref[...], w1_ref[...],
                              preferred_element_type=jnp.float32)


def _dense1(xp, xc, w1, degp):
    return pl.pallas_call(
        _dense1_body,
        grid=(NB,),
        in_specs=[
            pl.BlockSpec((BR, D), lambda r: (r, 0)),
            pl.BlockSpec((BR, D), lambda r: (r, 0)),
            pl.BlockSpec((D, D), lambda r: (0, 0)),
            pl.BlockSpec((BR, 16), lambda r: (r, 0)),
        ],
        out_specs=pl.BlockSpec((2, BR, D), lambda r: (0, r, 0)),
        out_shape=jax.ShapeDtypeStruct((2, NP, D), jnp.float32),
    )(xp, xc, w1, degp)


def _dense2_body(s1_ref, t1_ref, deg_ref, w2_ref, b1_ref, t2_ref):
    dis = _dis_from(deg_ref)
    z = jnp.maximum(dis * (s1_ref[0] + t1_ref[0]) + b1_ref[...], 0.0)
    t2_ref[0] = dis * jnp.dot(z, w2_ref[...], preferred_element_type=jnp.float32)


def _dense2(s1, t1, degp, w2, b1):
    return pl.pallas_call(
        _dense2_body,
        grid=(2, NB),
        in_specs=[
            pl.BlockSpec((1, BR, D), lambda g, r: (g, r, 0)),
            pl.BlockSpec((1, BR, D), lambda g, r: (g, r, 0)),
            pl.BlockSpec((BR, 16), lambda g, r: (r, 0)),
            pl.BlockSpec((D, D), lambda g, r: (0, 0)),
            pl.BlockSpec((1, D), lambda g, r: (0, 0)),
        ],
        out_specs=pl.BlockSpec((1, BR, D), lambda g, r: (g, r, 0)),
        out_shape=jax.ShapeDtypeStruct((2, NP, D), jnp.float32),
    )(s1, t1, degp, w2, b1)


def _final_body(s2_ref, t2_ref, deg_ref, b2_ref, out_ref, s_ref, acc_ref):
    g = pl.program_id(0)
    r = pl.program_id(1)
    dis = _dis_from(deg_ref)
    out = dis * (s2_ref[0] + t2_ref[0]) + b2_ref[...]
    out_ref[0] = out

    @pl.when(jnp.logical_and(g == 0, r == 0))
    def _():
        acc_ref[...] = jnp.zeros((8, D), jnp.float32)

    @pl.when(g == 0)
    def _():
        gid = r * BR + lax.broadcasted_iota(jnp.int32, (BR, 1), 0)
        masked = jnp.where(gid < N, out, 0.0)
        acc_ref[...] += masked.reshape(BR // 8, 8, D).sum(axis=0)

    @pl.when(jnp.logical_and(g == 0, r == NB - 1))
    def _():
        total = acc_ref[...].sum(axis=0, keepdims=True)
        mean = jnp.broadcast_to(total / float(N), (8, D))
        s_ref[...] = 1.0 / (1.0 + jnp.exp(-mean))


def _final(s2, t2, degp, b2):
    return pl.pallas_call(
        _final_body,
        grid=(2, NB),
        in_specs=[
            pl.BlockSpec((1, BR, D), lambda g, r: (g, r, 0)),
            pl.BlockSpec((1, BR, D), lambda g, r: (g, r, 0)),
            pl.BlockSpec((BR, 16), lambda g, r: (r, 0)),
            pl.BlockSpec((1, D), lambda g, r: (0, 0)),
        ],
        out_specs=[
            pl.BlockSpec((1, BR, D), lambda g, r: (g, r, 0)),
            pl.BlockSpec((8, D), lambda g, r: (0, 0)),
        ],
        out_shape=[
            jax.ShapeDtypeStruct((2, NP, D), jnp.float32),
            jax.ShapeDtypeStruct((8, D), jnp.float32),
        ],
        scratch_shapes=[pltpu.VMEM((8, D), jnp.float32)],
    )(s2, t2, degp, b2)


def kernel(x, edge_index, W1, b1, W2, b2):
    src = edge_index[0].astype(jnp.int32)
    dst = edge_index[1].astype(jnp.int32)

    perm = jax.random.permutation(jax.random.key(42), N).astype(jnp.int32)
    permfull = jnp.full((NP,), N, jnp.int32).at[:N].set(perm)
    permidx = jnp.pad(permfull.reshape(NC * NS, 5, 64), ((0, 0), (0, 3), (0, 0)),
                      constant_values=N)
    xp = jnp.zeros((NP, D), jnp.float32).at[:N].set(x)

    srcb = jnp.pad(src.reshape(NS, E // NS), ((0, 0), (0, KC * CH - E // NS)),
                   constant_values=DUMMY).reshape(NS, KC, CH)
    srcidx = jnp.stack([srcb, srcb + NP]).reshape(NC * NS * KC, CH)
    dstidx = jnp.pad(dst.reshape(NS, E // NS), ((0, 0), (0, KC * CH - E // NS)),
                     constant_values=DUMMY).reshape(NS * KC, CH)

    ones_table = jnp.zeros((NC * NP, D), jnp.float32).at[:, 0].set(1.0)
    srcidx0 = jnp.full_like(srcidx, DUMMY)
    s0 = _spmm_kernel(ones_table, srcidx0, dstidx)
    degc = s0[:NP, :16]

    xc = _xcor_kernel(permidx, xp)
    t1 = _dense1(xp, xc, W1, degc)
    s1 = _spmm_kernel(t1.reshape(2 * NP, D), srcidx, dstidx)
    t2 = _dense2(s1.reshape(2, NP, D), t1, degc, W2, b1.reshape(1, D))
    s2 = _spmm_kernel(t2.reshape(2 * NP, D), srcidx, dstidx)
    out2, sraw = _final(s2.reshape(2, NP, D), t2, degc, b2.reshape(1, D))

    pos_z = out2[0, :N]
    neg_z = out2[1, :N]
    s = sraw[0]
    return pos_z, neg_z, s
